# baseline (speedup 1.0000x reference)
import jax
import jax.numpy as jnp
from jax.experimental import pallas as pl
from jax.experimental.pallas import tpu as pltpu

_CHUNK_ROWS = 2048
_NBUF = 6


def _ring_body(x_ref, out_ref, mask_ref, buf, insem, outsem):
    n = x_ref.shape[0]
    c = min(_CHUNK_ROWS, n)
    nch = n // c

    def in_copy(j):
        b = j % _NBUF
        return pltpu.make_async_copy(
            x_ref.at[pl.ds(j * c, c)], buf.at[b], insem.at[b])

    def out_copy(j):
        b = j % _NBUF
        return pltpu.make_async_copy(
            buf.at[b], out_ref.at[pl.ds(j * c, c)], outsem.at[b])

    for j in range(min(_NBUF, nch)):
        in_copy(j).start()
    mask_ref[...] = jnp.ones(mask_ref.shape, dtype=jnp.bool_)
    for i in range(nch):
        if i > 0 and (i - 1) + _NBUF < nch:
            out_copy(i - 1).wait()
            in_copy((i - 1) + _NBUF).start()
        in_copy(i).wait()
        out_copy(i).start()
    for i in range(max(0, nch - _NBUF), nch):
        out_copy(i).wait()


def kernel(bags):
    b, s, d = bags.shape
    n = b * s
    flat = bags.reshape(n, d)
    c = min(_CHUNK_ROWS, n)
    padded, mask = pl.pallas_call(
        _ring_body,
        in_specs=[pl.BlockSpec(memory_space=pl.ANY)],
        out_specs=(
            pl.BlockSpec(memory_space=pl.ANY),
            pl.BlockSpec(memory_space=pltpu.MemorySpace.VMEM),
        ),
        out_shape=(
            jax.ShapeDtypeStruct((n, d), bags.dtype),
            jax.ShapeDtypeStruct((b, s), jnp.bool_),
        ),
        scratch_shapes=[
            pltpu.VMEM((_NBUF, c, d), bags.dtype),
            pltpu.SemaphoreType.DMA((_NBUF,)),
            pltpu.SemaphoreType.DMA((_NBUF,)),
        ],
    )(flat)
    return (padded.reshape(b, s, d), mask)


# speedup vs baseline: 1.0039x; 1.0039x over previous
import jax
import jax.numpy as jnp
from jax.experimental import pallas as pl
from jax.experimental.pallas import tpu as pltpu

_CHUNK_ROWS = 2048
_NBUF = 8
_WSLACK = 3


def _ring_body(x_ref, out_ref, mask_ref, buf, insem, outsem):
    n = x_ref.shape[0]
    c = min(_CHUNK_ROWS, n)
    nch = n // c

    def in_copy(j):
        b = j % _NBUF
        return pltpu.make_async_copy(
            x_ref.at[pl.ds(j * c, c)], buf.at[b], insem.at[b])

    def out_copy(j):
        b = j % _NBUF
        return pltpu.make_async_copy(
            buf.at[b], out_ref.at[pl.ds(j * c, c)], outsem.at[b])

    for j in range(min(_NBUF, nch)):
        in_copy(j).start()
    mask_ref[...] = jnp.ones(mask_ref.shape, dtype=jnp.bool_)
    for i in range(nch):
        in_copy(i).wait()
        out_copy(i).start()
        if i >= _WSLACK and (i - _WSLACK) + _NBUF < nch:
            out_copy(i - _WSLACK).wait()
            in_copy((i - _WSLACK) + _NBUF).start()
    for i in range(max(0, nch - _NBUF), nch):
        out_copy(i).wait()


def kernel(bags):
    b, s, d = bags.shape
    n = b * s
    flat = bags.reshape(n, d)
    c = min(_CHUNK_ROWS, n)
    padded, mask = pl.pallas_call(
        _ring_body,
        in_specs=[pl.BlockSpec(memory_space=pl.ANY)],
        out_specs=(
            pl.BlockSpec(memory_space=pl.ANY),
            pl.BlockSpec(memory_space=pltpu.MemorySpace.VMEM),
        ),
        out_shape=(
            jax.ShapeDtypeStruct((n, d), bags.dtype),
            jax.ShapeDtypeStruct((b, s), jnp.bool_),
        ),
        scratch_shapes=[
            pltpu.VMEM((_NBUF, c, d), bags.dtype),
            pltpu.SemaphoreType.DMA((_NBUF,)),
            pltpu.SemaphoreType.DMA((_NBUF,)),
        ],
    )(flat)
    return (padded.reshape(b, s, d), mask)
